# Initial kernel scaffold; baseline (speedup 1.0000x reference)
import jax
import jax.numpy as jnp
from jax.experimental import pallas as pl


def kernel(x, table):
    raise NotImplementedError("write your pallas kernel here")


# speedup vs baseline: 1.0228x; 1.0228x over previous
import functools

import jax
import jax.numpy as jnp
from jax import lax
from jax.experimental import pallas as pl
from jax.experimental.pallas import tpu as pltpu
from jax.experimental.pallas import tpu_sc as plsc


def kernel(x, table):
    B, S = x.shape
    V, D = table.shape
    N = B * S

    NW = 32
    G = 128
    n_per_w = N // NW
    n_groups = n_per_w // G

    idx = x.reshape(NW, n_groups, G).astype(jnp.int32)

    mesh = plsc.VectorSubcoreMesh(core_axis_name="c", subcore_axis_name="s")

    @functools.partial(
        pl.kernel,
        mesh=mesh,
        out_type=jax.ShapeDtypeStruct((N, D), jnp.float32),
        compiler_params=pltpu.CompilerParams(use_tc_tiling_on_sc=False),
        scratch_types=[
            pltpu.VMEM((n_groups, G), jnp.int32),
            pltpu.VMEM((G, D), jnp.float32),
            pltpu.SemaphoreType.DMA,
        ],
    )
    def emb(idx_hbm, table_hbm, out_hbm, idx_v, rows_v, sem):
        wid = lax.axis_index("s") * 2 + lax.axis_index("c")
        base = wid * n_per_w
        pltpu.sync_copy(idx_hbm.at[wid], idx_v)

        def body(g, carry):
            off = base + g * G
            pltpu.async_copy(table_hbm.at[idx_v.at[g]], rows_v, sem).wait()
            pltpu.sync_copy(rows_v, out_hbm.at[pl.ds(off, G)])
            return carry

        lax.fori_loop(0, n_groups, body, 0)

    out = emb(idx, table)
    return out.reshape(B, S, D)


# speedup vs baseline: 1.3049x; 1.2758x over previous
import functools

import jax
import jax.numpy as jnp
from jax import lax
from jax.experimental import pallas as pl
from jax.experimental.pallas import tpu as pltpu
from jax.experimental.pallas import tpu_sc as plsc


def kernel(x, table):
    B, S = x.shape
    V, D = table.shape
    N = B * S

    NW = 32
    G = 128
    K = 8
    T = N // (NW * G * K)
    n_groups = T * K

    idx = x.reshape(NW, n_groups, G).astype(jnp.int32)

    mesh = plsc.VectorSubcoreMesh(core_axis_name="c", subcore_axis_name="s")

    @functools.partial(
        pl.kernel,
        mesh=mesh,
        out_type=jax.ShapeDtypeStruct((NW * T, K, G, D), jnp.float32),
        compiler_params=pltpu.CompilerParams(use_tc_tiling_on_sc=False),
        scratch_types=[
            pltpu.VMEM((n_groups, G), jnp.int32),
            pltpu.VMEM((2, K, G, D), jnp.float32),
            pltpu.SemaphoreType.DMA,
            pltpu.SemaphoreType.DMA,
        ],
    )
    def emb(idx_hbm, table_hbm, out_hbm, idx_v, rows_v, gsem, osem):
        wid = lax.axis_index("s") * 2 + lax.axis_index("c")
        pltpu.sync_copy(idx_hbm.at[wid], idx_v)
        wbase = wid * T

        def body(t, carry):
            p = lax.rem(t, 2)

            @pl.when(t >= 2)
            def _wait_prev_out():
                pltpu.make_async_copy(rows_v.at[p], out_hbm.at[wbase], osem).wait()

            copies = [
                pltpu.async_copy(
                    table_hbm.at[idx_v.at[t * K + j]], rows_v.at[p, j], gsem)
                for j in range(K)
            ]
            for c in copies:
                c.wait()
            pltpu.async_copy(rows_v.at[p], out_hbm.at[wbase + t], osem)
            return carry

        lax.fori_loop(0, T, body, 0)
        pltpu.make_async_copy(rows_v.at[0], out_hbm.at[wbase], osem).wait()
        pltpu.make_async_copy(rows_v.at[1], out_hbm.at[wbase], osem).wait()

    out = emb(idx, table)
    return out.reshape(B, S, D)
